# baseline (speedup 1.0000x reference)
import jax
import jax.numpy as jnp
from jax import lax
from jax.experimental import pallas as pl
from jax.experimental.pallas import tpu as pltpu
from jax.experimental.pallas import tpu_sc as plsc

B = 4096
T = 192
F = 36
NUM = 10
NEMB = 26
V = 100000
D = 32
TIN = 168

NC, NS = 2, 16
NW = NC * NS
BPW = B // NW


def _make_emb_kernel(nj):
    def _emb_kernel(idx_hbm, tab_hbm, embt_hbm, idx_v, buf_v, acc_v, sem, gsem):
        wid = lax.axis_index("s") * NC + lax.axis_index("c")
        base = wid * BPW
        pltpu.sync_copy(idx_hbm.at[:, pl.ds(base, BPW)], idx_v)
        for j in range(nj):
            for d in range(D):
                pltpu.async_copy(
                    tab_hbm.at[pl.ds((j * D + d) * V, V)].at[idx_v.at[j]],
                    buf_v.at[j, d],
                    gsem,
                )
        pltpu.make_async_copy(
            tab_hbm.at[pl.ds(0, nj * D * BPW)], buf_v, gsem
        ).wait()

        def body(j, carry):
            for d in range(D):
                for c in range(BPW // 16):
                    s = pl.ds(c * 16, 16)
                    acc_v[d, s] = acc_v[d, s] + buf_v[j, d, s]
            return carry

        for d in range(D):
            for c in range(BPW // 16):
                s = pl.ds(c * 16, 16)
                acc_v[d, s] = buf_v[0, d, s]
        lax.fori_loop(1, nj, body, 0)
        pltpu.sync_copy(acc_v, embt_hbm.at[:, pl.ds(base, BPW)])

    return _emb_kernel


def _emb_part(idx_t, tab_flat, nj):
    mesh = plsc.VectorSubcoreMesh(core_axis_name="c", subcore_axis_name="s")
    f = pl.kernel(
        _make_emb_kernel(nj),
        out_type=jax.ShapeDtypeStruct((D, B), jnp.float32),
        mesh=mesh,
        scratch_types=[
            pltpu.VMEM((nj, BPW), jnp.int32),
            pltpu.VMEM((nj, D, BPW), jnp.float32),
            pltpu.VMEM((D, BPW), jnp.float32),
            pltpu.SemaphoreType.DMA,
            pltpu.SemaphoreType.DMA,
        ],
        compiler_params=pltpu.CompilerParams(use_tc_tiling_on_sc=False),
    )
    return f(idx_t, tab_flat)


def _idx_kernel(x_ref, idx_ref):
    idx_ref[...] = x_ref[NUM:, 0, :].astype(jnp.int32)


def _idx(xT):
    return pl.pallas_call(
        _idx_kernel,
        grid=(1,),
        in_specs=[pl.BlockSpec((F, 8, B), lambda i: (0, 0, 0))],
        out_specs=pl.BlockSpec((NEMB, B), lambda i: (0, 0)),
        out_shape=jax.ShapeDtypeStruct((NEMB, B), jnp.int32),
    )(xT)


_BL = 512


def _slice_kernel(x_ref, past_ref, fut_ref, mask_ref):
    past_ref[...] = x_ref[:NUM, :TIN, :]
    fut_ref[...] = x_ref[:NUM, TIN:, :]
    mask_ref[...] = (x_ref[0, :TIN, :] > 0).astype(jnp.float32)


def _slices(xT):
    return pl.pallas_call(
        _slice_kernel,
        grid=(B // _BL,),
        in_specs=[pl.BlockSpec((F, T, _BL), lambda i: (0, 0, i))],
        out_specs=(
            pl.BlockSpec((NUM, TIN, _BL), lambda i: (0, 0, i)),
            pl.BlockSpec((NUM, T - TIN, _BL), lambda i: (0, 0, i)),
            pl.BlockSpec((TIN, _BL), lambda i: (0, i)),
        ),
        out_shape=(
            jax.ShapeDtypeStruct((NUM, TIN, B), jnp.float32),
            jax.ShapeDtypeStruct((NUM, T - TIN, B), jnp.float32),
            jax.ShapeDtypeStruct((TIN, B), jnp.float32),
        ),
    )(xT)


def kernel(x, tables):
    xT = x.transpose(2, 1, 0)
    pastT, futT, maskT = _slices(xT)
    past = pastT.transpose(2, 1, 0)
    fut = futT.transpose(2, 1, 0)
    mask_out = maskT.T.reshape(B, 1, TIN).astype(jnp.bool_)

    idx_t = _idx(xT)
    ks = 2
    js = NEMB // ks
    embT = None
    for k in range(ks):
        tab_k = tables[k * js:(k + 1) * js].transpose(0, 2, 1)
        e = _emb_part(idx_t[k * js:(k + 1) * js], tab_k.reshape(js * D * V), js)
        embT = e if embT is None else embT + e
    emb = embT.T
    return (past, fut, emb, mask_out)


# speedup vs baseline: 1.2121x; 1.2121x over previous
import jax
import jax.numpy as jnp
from jax import lax
from jax.experimental import pallas as pl
from jax.experimental.pallas import tpu as pltpu
from jax.experimental.pallas import tpu_sc as plsc

B = 4096
T = 192
F = 36
NUM = 10
NEMB = 26
V = 100000
D = 32
TIN = 168

NC, NS = 2, 16
NW = NC * NS
BPW = B // NW


def _emb_kernel(idx_hbm, tab_hbm, embt_hbm, idx_v, buf_v, acc_v, sem, gsem):
    wid = lax.axis_index("s") * NC + lax.axis_index("c")
    base = wid * BPW
    pltpu.sync_copy(idx_hbm.at[:, pl.ds(base, BPW)], idx_v)
    for j in range(NEMB):
        for d in range(D):
            pltpu.async_copy(
                tab_hbm.at[j, d].at[idx_v.at[j]],
                buf_v.at[j, d],
                gsem,
            )
    for j in range(NEMB):
        pltpu.make_async_copy(
            tab_hbm.at[0, 0].at[pl.ds(0, D * BPW)], buf_v.at[j], gsem
        ).wait()

    def body(j, carry):
        for d in range(D):
            for c in range(BPW // 16):
                s = pl.ds(c * 16, 16)
                acc_v[d, s] = acc_v[d, s] + buf_v[j, d, s]
        return carry

    for d in range(D):
        for c in range(BPW // 16):
            s = pl.ds(c * 16, 16)
            acc_v[d, s] = buf_v[0, d, s]
    lax.fori_loop(1, NEMB, body, 0)
    pltpu.sync_copy(acc_v, embt_hbm.at[:, pl.ds(base, BPW)])


def _emb(idx_t, tab_flat):
    mesh = plsc.VectorSubcoreMesh(core_axis_name="c", subcore_axis_name="s")
    f = pl.kernel(
        _emb_kernel,
        out_type=jax.ShapeDtypeStruct((D, B), jnp.float32),
        mesh=mesh,
        scratch_types=[
            pltpu.VMEM((NEMB, BPW), jnp.int32),
            pltpu.VMEM((NEMB, D, BPW), jnp.float32),
            pltpu.VMEM((D, BPW), jnp.float32),
            pltpu.SemaphoreType.DMA,
            pltpu.SemaphoreType.DMA,
        ],
        compiler_params=pltpu.CompilerParams(use_tc_tiling_on_sc=False),
    )
    return f(idx_t, tab_flat)


def _idx_kernel(x_ref, idx_ref):
    idx_ref[...] = x_ref[NUM:, 0, :].astype(jnp.int32)


def _idx(xT):
    return pl.pallas_call(
        _idx_kernel,
        grid=(1,),
        in_specs=[pl.BlockSpec((F, 8, B), lambda i: (0, 0, 0))],
        out_specs=pl.BlockSpec((NEMB, B), lambda i: (0, 0)),
        out_shape=jax.ShapeDtypeStruct((NEMB, B), jnp.int32),
    )(xT)


_BL = 512


def _slice_kernel(x_ref, past_ref, fut_ref, mask_ref):
    past_ref[...] = x_ref[:NUM, :TIN, :]
    fut_ref[...] = x_ref[:NUM, TIN:, :]
    mask_ref[...] = (x_ref[0, :TIN, :] > 0).astype(jnp.float32)


def _slices(xT):
    return pl.pallas_call(
        _slice_kernel,
        grid=(B // _BL,),
        in_specs=[pl.BlockSpec((F, T, _BL), lambda i: (0, 0, i))],
        out_specs=(
            pl.BlockSpec((NUM, TIN, _BL), lambda i: (0, 0, i)),
            pl.BlockSpec((NUM, T - TIN, _BL), lambda i: (0, 0, i)),
            pl.BlockSpec((TIN, _BL), lambda i: (0, i)),
        ),
        out_shape=(
            jax.ShapeDtypeStruct((NUM, TIN, B), jnp.float32),
            jax.ShapeDtypeStruct((NUM, T - TIN, B), jnp.float32),
            jax.ShapeDtypeStruct((TIN, B), jnp.float32),
        ),
    )(xT)


def kernel(x, tables):
    xT = x.transpose(2, 1, 0)
    pastT, futT, maskT = _slices(xT)
    past = pastT.transpose(2, 1, 0)
    fut = futT.transpose(2, 1, 0)
    mask_out = maskT.T.reshape(B, 1, TIN).astype(jnp.bool_)

    idx_t = _idx(xT)
    tab_t = tables.transpose(0, 2, 1)
    embT = _emb(idx_t, tab_t)
    emb = embT.T
    return (past, fut, emb, mask_out)
